# Initial kernel scaffold; baseline (speedup 1.0000x reference)
import jax
import jax.numpy as jnp
from jax.experimental import pallas as pl


def kernel(x, embedding):
    raise NotImplementedError("write your pallas kernel here")


# speedup vs baseline: 1.7232x; 1.7232x over previous
import jax
import jax.numpy as jnp
from jax.experimental import pallas as pl


def _add_block(x_ref, e_ref, o_ref):
    o_ref[...] = x_ref[...] + e_ref[...][None, :, :]


def kernel(x, embedding):
    batch, seq_len, d_model = x.shape
    s_blk = 256
    while seq_len % s_blk:
        s_blk //= 2
    grid = (seq_len // s_blk,)
    return pl.pallas_call(
        _add_block,
        grid=grid,
        in_specs=[
            pl.BlockSpec((batch, s_blk, d_model), lambda i: (0, i, 0)),
            pl.BlockSpec((s_blk, d_model), lambda i: (i, 0)),
        ],
        out_specs=pl.BlockSpec((batch, s_blk, d_model), lambda i: (0, i, 0)),
        out_shape=jax.ShapeDtypeStruct(x.shape, x.dtype),
    )(x, embedding)
